# baseline (speedup 1.0000x reference)
import jax
import jax.numpy as jnp
from jax.experimental import pallas as pl

_R = 512


def _mpnn_block(x_ref, adjT_ref, wmsg_ref, wnode_ref, out_ref):
    j = pl.program_id(0)
    r = out_ref.shape[0]
    n = x_ref.shape[0]
    a = adjT_ref[...].astype(jnp.bfloat16)
    xb = x_ref[...].astype(jnp.bfloat16)
    msgsum = jnp.dot(a, xb, preferred_element_type=jnp.float32)
    ones = jnp.ones((n, 1), jnp.bfloat16)
    deg = jnp.dot(a, ones, preferred_element_type=jnp.float32)
    msg = (msgsum * (1.0 / jnp.maximum(deg, 1.0))).astype(jnp.bfloat16)
    xblk = x_ref[pl.ds(j * r, r), :].astype(jnp.bfloat16)
    wnode = wnode_ref[...].astype(jnp.bfloat16)
    wmsg = wmsg_ref[...].astype(jnp.bfloat16)
    node = jnp.dot(xblk, wnode, preferred_element_type=jnp.float32)
    msg2 = jnp.dot(msg, wmsg, preferred_element_type=jnp.float32)
    out_ref[...] = jnp.maximum(node + msg2, 0.0)


def kernel(x, adj, W_msg, W_node):
    B, N, D = x.shape
    U = W_msg.shape[1]
    x2d = x.reshape(N, D)
    adjT = adj.reshape(N, N).view(jnp.int8).T

    out = pl.pallas_call(
        _mpnn_block,
        grid=(N // _R,),
        in_specs=[
            pl.BlockSpec((N, D), lambda j: (0, 0)),
            pl.BlockSpec((_R, N), lambda j: (j, 0)),
            pl.BlockSpec((D, U), lambda j: (0, 0)),
            pl.BlockSpec((D, U), lambda j: (0, 0)),
        ],
        out_specs=pl.BlockSpec((_R, U), lambda j: (j, 0)),
        out_shape=jax.ShapeDtypeStruct((N, U), jnp.float32),
    )(x2d, adjT, W_msg, W_node)
    return out.reshape(B, N, U)
